# baseline (speedup 1.0000x reference)
import jax
import jax.numpy as jnp
from jax import lax
from jax.experimental import pallas as pl
from jax.experimental.pallas import tpu as pltpu
from jax.experimental.pallas import tpu_sc as plsc

_N, _E, _D = 10000, 320000, 128
_NC, _NS = 2, 16
_NW = _NC * _NS
_EPW = _E // _NW
_C = 80
_NCHUNK = 128
_CAP = _NCHUNK * _C
_RPT = 632
_NPAD = _NS * _RPT
_CNT_PAD = _NS * 640
_F32 = jnp.float32


def _build_agg(with_count):
    mesh = plsc.VectorSubcoreMesh(core_axis_name="c", subcore_axis_name="s")
    out_type = [jax.ShapeDtypeStruct((_NC, _NPAD, _D), _F32)]
    scratch = [
        pltpu.VMEM((_C,), jnp.int32),
        pltpu.VMEM((_C,), jnp.int32),
        pltpu.VMEM((_C, _D), _F32),
        pltpu.VMEM_SHARED((_NPAD, _D), _F32),
        pltpu.SemaphoreType.DMA,
    ]
    if with_count:
        out_type.append(jax.ShapeDtypeStruct((_NC, _CNT_PAD), _F32))
        scratch += [
            pltpu.VMEM((_C,), _F32),
            pltpu.VMEM((640,), _F32),
            pltpu.VMEM_SHARED((_CNT_PAD,), _F32),
            pltpu.SemaphoreType.DMA,
        ]

    def body(y_hbm, src_hbm, dst_hbm, agg_out, *rest):
        if with_count:
            (cnt_out, src_v, dst_v, rows_v, acc_sp, sg,
             ones_v, zc_v, cnt_sp, sc_sem) = rest
        else:
            src_v, dst_v, rows_v, acc_sp, sg = rest

        core = lax.axis_index("c")
        sub = lax.axis_index("s")
        wid = core * _NS + sub

        @pl.loop(0, _C)
        def _(i):
            for j in range(_D // 16):
                rows_v[i, pl.ds(j * 16, 16)] = jnp.zeros((16,), _F32)

        a0 = sub * _RPT
        nfull = _RPT // _C
        rem = _RPT % _C
        for k in range(nfull):
            pltpu.sync_copy(rows_v, acc_sp.at[pl.ds(a0 + k * _C, _C)])
        if rem:
            pltpu.sync_copy(rows_v.at[pl.ds(0, rem)],
                            acc_sp.at[pl.ds(a0 + nfull * _C, rem)])

        if with_count:
            @pl.loop(0, 640 // 16)
            def _(i):
                zc_v[pl.ds(i * 16, 16)] = jnp.zeros((16,), _F32)

            @pl.loop(0, _C // 16)
            def _(i):
                ones_v[pl.ds(i * 16, 16)] = jnp.ones((16,), _F32)

            pltpu.sync_copy(zc_v, cnt_sp.at[pl.ds(sub * 640, 640)])

        plsc.subcore_barrier()

        ebase = wid * _CAP

        @pl.loop(0, _NCHUNK)
        def _(j):
            b = ebase + j * _C
            pltpu.sync_copy(src_hbm.at[pl.ds(b, _C)], src_v)
            pltpu.sync_copy(dst_hbm.at[pl.ds(b, _C)], dst_v)
            pltpu.async_copy(y_hbm.at[src_v], rows_v, sg).wait()
            pltpu.sync_copy(rows_v, acc_sp.at[dst_v], add=True)
            if with_count:
                pltpu.async_copy(ones_v, cnt_sp.at[dst_v],
                                 sc_sem, add=True)

        if with_count:
            @pl.loop(0, _NCHUNK)
            def _(j):
                pltpu.make_async_copy(ones_v, cnt_sp.at[dst_v],
                                      sc_sem).wait()

        plsc.subcore_barrier()

        for k in range(nfull):
            sl = pl.ds(a0 + k * _C, _C)
            pltpu.sync_copy(acc_sp.at[sl], agg_out.at[core].at[sl])
        if rem:
            sl = pl.ds(a0 + nfull * _C, rem)
            pltpu.sync_copy(acc_sp.at[sl], agg_out.at[core].at[sl])
        if with_count:
            sl = pl.ds(sub * 640, 640)
            pltpu.sync_copy(cnt_sp.at[sl], cnt_out.at[core].at[sl])

    return pl.kernel(body, out_type=out_type, mesh=mesh, scratch_types=scratch)


_agg_with_count = _build_agg(True)
_agg_no_count = _build_agg(False)

_BM = 2000


def _sage_tc(xin, p, cnt, Wl, bl, Wr, relu):
    def body(x_ref, p_ref, c_ref, wl_ref, b_ref, wr_ref, o_ref):
        s = p_ref[0] + p_ref[1]
        c = jnp.maximum(c_ref[0] + c_ref[1], 1.0)
        mean = s / c
        acc = lax.dot_general(mean, wl_ref[...], (((1,), (1,)), ((), ())),
                              preferred_element_type=_F32)
        acc = acc + b_ref[...]
        acc = acc + lax.dot_general(x_ref[...], wr_ref[...],
                                    (((1,), (1,)), ((), ())),
                                    preferred_element_type=_F32)
        if relu:
            acc = jnp.maximum(acc, 0.0)
        o_ref[...] = acc

    return pl.pallas_call(
        body,
        grid=(_N // _BM,),
        in_specs=[
            pl.BlockSpec((_BM, _D), lambda i: (i, 0)),
            pl.BlockSpec((_NC, _BM, _D), lambda i: (0, i, 0)),
            pl.BlockSpec((_NC, _BM, 1), lambda i: (0, i, 0)),
            pl.BlockSpec((_D, _D), lambda i: (0, 0)),
            pl.BlockSpec((1, _D), lambda i: (0, 0)),
            pl.BlockSpec((_D, _D), lambda i: (0, 0)),
        ],
        out_specs=pl.BlockSpec((_BM, _D), lambda i: (i, 0)),
        out_shape=jax.ShapeDtypeStruct((_N, _D), _F32),
    )(xin, p, cnt, Wl, bl, Wr)


def _pack_edges(edge_index):
    src = edge_index[0].astype(jnp.int32).reshape(_NW, _EPW)
    dst = edge_index[1].astype(jnp.int32).reshape(_NW, _EPW)
    pad = _CAP - _EPW
    srcp = jnp.concatenate(
        [src, jnp.zeros((_NW, pad), jnp.int32)], axis=1)
    dstp = jnp.concatenate(
        [dst, jnp.full((_NW, pad), _N, jnp.int32)], axis=1)
    return srcp.reshape(_NW * _CAP), dstp.reshape(_NW * _CAP)


def kernel(x, edge_index, Wl0, bl0, Wr0, Wl1, bl1, Wr1):
    srcf, dstf = _pack_edges(edge_index)
    bl0r = bl0.reshape(1, _D)
    bl1r = bl1.reshape(1, _D)

    agg1, cnt_raw = _agg_with_count(x, srcf, dstf)
    cnt = cnt_raw[:, :_N].reshape(_NC, _N, 1)
    h = _sage_tc(x, agg1[:, :_N], cnt, Wl0, bl0r, Wr0, relu=True)
    (agg2,) = _agg_no_count(h, srcf, dstf)
    out = _sage_tc(h, agg2[:, :_N], cnt, Wl1, bl1r, Wr1, relu=False)
    return out


# speedup vs baseline: 1.1126x; 1.1126x over previous
import jax
import jax.numpy as jnp
from jax import lax
from jax.experimental import pallas as pl
from jax.experimental.pallas import tpu as pltpu
from jax.experimental.pallas import tpu_sc as plsc

_N, _E, _D = 10000, 320000, 128
_NC, _NS = 2, 16
_NW = _NC * _NS
_EPW = _E // _NW
_C = 128
_NCHUNK = 80
_CAP = _NCHUNK * _C
_RPT = 632
_NPAD = _NS * _RPT
_CNT_PAD = _NS * 640
_F32 = jnp.float32


def _build_agg(with_count):
    mesh = plsc.VectorSubcoreMesh(core_axis_name="c", subcore_axis_name="s")
    out_type = [jax.ShapeDtypeStruct((_NC, _NPAD, _D), _F32)]
    scratch = [
        pltpu.VMEM((_C,), jnp.int32),
        pltpu.VMEM((_C,), jnp.int32),
        pltpu.VMEM((_C, _D), _F32),
        pltpu.VMEM_SHARED((_NPAD, _D), _F32),
        pltpu.SemaphoreType.DMA,
    ]
    if with_count:
        out_type.append(jax.ShapeDtypeStruct((_NC, _CNT_PAD), _F32))
        scratch += [
            pltpu.VMEM((_C,), _F32),
            pltpu.VMEM((640,), _F32),
            pltpu.VMEM_SHARED((_CNT_PAD,), _F32),
            pltpu.SemaphoreType.DMA,
        ]

    def body(y_hbm, src_hbm, dst_hbm, agg_out, *rest):
        if with_count:
            (cnt_out, src_v, dst_v, rows_v, acc_sp, sg,
             ones_v, zc_v, cnt_sp, sc_sem) = rest
        else:
            src_v, dst_v, rows_v, acc_sp, sg = rest

        core = lax.axis_index("c")
        sub = lax.axis_index("s")
        wid = core * _NS + sub

        @pl.loop(0, _C)
        def _(i):
            for j in range(_D // 16):
                rows_v[i, pl.ds(j * 16, 16)] = jnp.zeros((16,), _F32)

        a0 = sub * _RPT
        nfull = _RPT // _C
        rem = _RPT % _C
        for k in range(nfull):
            pltpu.sync_copy(rows_v, acc_sp.at[pl.ds(a0 + k * _C, _C)])
        if rem:
            pltpu.sync_copy(rows_v.at[pl.ds(0, rem)],
                            acc_sp.at[pl.ds(a0 + nfull * _C, rem)])

        if with_count:
            @pl.loop(0, 640 // 16)
            def _(i):
                zc_v[pl.ds(i * 16, 16)] = jnp.zeros((16,), _F32)

            @pl.loop(0, _C // 16)
            def _(i):
                ones_v[pl.ds(i * 16, 16)] = jnp.ones((16,), _F32)

            pltpu.sync_copy(zc_v, cnt_sp.at[pl.ds(sub * 640, 640)])

        plsc.subcore_barrier()

        ebase = wid * _CAP

        @pl.loop(0, _NCHUNK)
        def _(j):
            b = ebase + j * _C
            pltpu.sync_copy(src_hbm.at[pl.ds(b, _C)], src_v)
            pltpu.sync_copy(dst_hbm.at[pl.ds(b, _C)], dst_v)
            pltpu.async_copy(y_hbm.at[src_v], rows_v, sg).wait()
            pltpu.sync_copy(rows_v, acc_sp.at[dst_v], add=True)
            if with_count:
                pltpu.async_copy(ones_v, cnt_sp.at[dst_v],
                                 sc_sem, add=True)

        if with_count:
            @pl.loop(0, _NCHUNK)
            def _(j):
                pltpu.make_async_copy(ones_v, cnt_sp.at[dst_v],
                                      sc_sem).wait()

        plsc.subcore_barrier()

        for k in range(nfull):
            sl = pl.ds(a0 + k * _C, _C)
            pltpu.sync_copy(acc_sp.at[sl], agg_out.at[core].at[sl])
        if rem:
            sl = pl.ds(a0 + nfull * _C, rem)
            pltpu.sync_copy(acc_sp.at[sl], agg_out.at[core].at[sl])
        if with_count:
            sl = pl.ds(sub * 640, 640)
            pltpu.sync_copy(cnt_sp.at[sl], cnt_out.at[core].at[sl])

    return pl.kernel(body, out_type=out_type, mesh=mesh, scratch_types=scratch)


_agg_with_count = _build_agg(True)
_agg_no_count = _build_agg(False)

_BM = 2000


def _sage_tc(xin, p, cnt, Wl, bl, Wr, relu):
    def body(x_ref, p_ref, c_ref, wl_ref, b_ref, wr_ref, o_ref):
        s = p_ref[0] + p_ref[1]
        c = jnp.maximum(c_ref[0] + c_ref[1], 1.0)
        mean = s / c
        acc = lax.dot_general(mean, wl_ref[...], (((1,), (1,)), ((), ())),
                              preferred_element_type=_F32)
        acc = acc + b_ref[...]
        acc = acc + lax.dot_general(x_ref[...], wr_ref[...],
                                    (((1,), (1,)), ((), ())),
                                    preferred_element_type=_F32)
        if relu:
            acc = jnp.maximum(acc, 0.0)
        o_ref[...] = acc

    return pl.pallas_call(
        body,
        grid=(_N // _BM,),
        in_specs=[
            pl.BlockSpec((_BM, _D), lambda i: (i, 0)),
            pl.BlockSpec((_NC, _BM, _D), lambda i: (0, i, 0)),
            pl.BlockSpec((_NC, _BM, 1), lambda i: (0, i, 0)),
            pl.BlockSpec((_D, _D), lambda i: (0, 0)),
            pl.BlockSpec((1, _D), lambda i: (0, 0)),
            pl.BlockSpec((_D, _D), lambda i: (0, 0)),
        ],
        out_specs=pl.BlockSpec((_BM, _D), lambda i: (i, 0)),
        out_shape=jax.ShapeDtypeStruct((_N, _D), _F32),
    )(xin, p, cnt, Wl, bl, Wr)


def _pack_edges(edge_index):
    src = edge_index[0].astype(jnp.int32).reshape(_NW, _EPW)
    dst = edge_index[1].astype(jnp.int32).reshape(_NW, _EPW)
    pad = _CAP - _EPW
    srcp = jnp.concatenate(
        [src, jnp.zeros((_NW, pad), jnp.int32)], axis=1)
    padrow = _N + (jnp.arange(_NW * pad, dtype=jnp.int32) % (_NPAD - _N))
    dstp = jnp.concatenate(
        [dst, padrow.reshape(_NW, pad)], axis=1)
    return srcp.reshape(_NW * _CAP), dstp.reshape(_NW * _CAP)


def kernel(x, edge_index, Wl0, bl0, Wr0, Wl1, bl1, Wr1):
    srcf, dstf = _pack_edges(edge_index)
    bl0r = bl0.reshape(1, _D)
    bl1r = bl1.reshape(1, _D)

    agg1, cnt_raw = _agg_with_count(x, srcf, dstf)
    cnt = cnt_raw[:, :_N].reshape(_NC, _N, 1)
    h = _sage_tc(x, agg1[:, :_N], cnt, Wl0, bl0r, Wr0, relu=True)
    (agg2,) = _agg_no_count(h, srcf, dstf)
    out = _sage_tc(h, agg2[:, :_N], cnt, Wl1, bl1r, Wr1, relu=False)
    return out


# speedup vs baseline: 1.7869x; 1.6061x over previous
import jax
import jax.numpy as jnp
from jax import lax
from jax.experimental import pallas as pl
from jax.experimental.pallas import tpu as pltpu
from jax.experimental.pallas import tpu_sc as plsc

_N, _E, _D = 10000, 320000, 128
_NC, _NS = 2, 16
_NW = _NC * _NS
_EPW = _E // _NW
_C = 80
_NCHUNK = _EPW // _C
_RPT = 632
_NPAD = _NS * _RPT
_CNT_PAD = _NS * 640
_F32 = jnp.float32


def _build_agg(with_count):
    mesh = plsc.VectorSubcoreMesh(core_axis_name="c", subcore_axis_name="s")
    out_type = [jax.ShapeDtypeStruct((_NC, _NPAD, _D), _F32)]
    scratch = [
        pltpu.VMEM((_C,), jnp.int32),
        pltpu.VMEM((_C,), jnp.int32),
        pltpu.VMEM((_C, _D), _F32),
        pltpu.VMEM_SHARED((_NPAD, _D), _F32),
        pltpu.SemaphoreType.DMA,
    ]
    if with_count:
        out_type.append(jax.ShapeDtypeStruct((_NC, _CNT_PAD), _F32))
        scratch += [
            pltpu.VMEM((_C,), _F32),
            pltpu.VMEM((640,), _F32),
            pltpu.VMEM_SHARED((_CNT_PAD,), _F32),
        ]

    def body(y_hbm, src_hbm, dst_hbm, agg_out, *rest):
        if with_count:
            cnt_out, src_v, dst_v, rows_v, acc_sp, sem, ones_v, zc_v, cnt_sp = rest
        else:
            src_v, dst_v, rows_v, acc_sp, sem = rest

        core = lax.axis_index("c")
        sub = lax.axis_index("s")
        wid = core * _NS + sub

        @pl.loop(0, _C)
        def _(i):
            for j in range(_D // 16):
                rows_v[i, pl.ds(j * 16, 16)] = jnp.zeros((16,), _F32)

        r0 = sub * _RPT
        nfull = _RPT // _C
        rem = _RPT % _C
        for k in range(nfull):
            pltpu.sync_copy(rows_v, acc_sp.at[pl.ds(r0 + k * _C, _C)])
        if rem:
            pltpu.sync_copy(rows_v.at[pl.ds(0, rem)],
                            acc_sp.at[pl.ds(r0 + nfull * _C, rem)])

        if with_count:
            @pl.loop(0, 640 // 16)
            def _(i):
                zc_v[pl.ds(i * 16, 16)] = jnp.zeros((16,), _F32)

            @pl.loop(0, _C // 16)
            def _(i):
                ones_v[pl.ds(i * 16, 16)] = jnp.ones((16,), _F32)

            pltpu.sync_copy(zc_v, cnt_sp.at[pl.ds(sub * 640, 640)])

        plsc.subcore_barrier()

        ebase = wid * _EPW

        @pl.loop(0, _NCHUNK)
        def _(j):
            b = ebase + j * _C
            pltpu.sync_copy(src_hbm.at[pl.ds(b, _C)], src_v)
            pltpu.sync_copy(dst_hbm.at[pl.ds(b, _C)], dst_v)
            pltpu.async_copy(y_hbm.at[src_v], rows_v, sem).wait()
            pltpu.sync_copy(rows_v, acc_sp.at[dst_v], add=True)
            if with_count:
                pltpu.sync_copy(ones_v, cnt_sp.at[dst_v], add=True)

        plsc.subcore_barrier()

        for k in range(nfull):
            sl = pl.ds(r0 + k * _C, _C)
            pltpu.sync_copy(acc_sp.at[sl], agg_out.at[core].at[sl])
        if rem:
            sl = pl.ds(r0 + nfull * _C, rem)
            pltpu.sync_copy(acc_sp.at[sl], agg_out.at[core].at[sl])
        if with_count:
            sl = pl.ds(sub * 640, 640)
            pltpu.sync_copy(cnt_sp.at[sl], cnt_out.at[core].at[sl])

    return pl.kernel(body, out_type=out_type, mesh=mesh, scratch_types=scratch)


_agg_with_count = _build_agg(True)
_agg_no_count = _build_agg(False)

_BM = 2000


def _sage_tc(xin, p, cnt, Wl, bl, Wr, relu):
    def body(x_ref, p_ref, c_ref, wl_ref, b_ref, wr_ref, o_ref):
        s = p_ref[0] + p_ref[1]
        c = jnp.maximum(c_ref[0] + c_ref[1], 1.0)
        mean = s / c
        acc = lax.dot_general(mean, wl_ref[...], (((1,), (1,)), ((), ())),
                              preferred_element_type=_F32)
        acc = acc + b_ref[...]
        acc = acc + lax.dot_general(x_ref[...], wr_ref[...],
                                    (((1,), (1,)), ((), ())),
                                    preferred_element_type=_F32)
        if relu:
            acc = jnp.maximum(acc, 0.0)
        o_ref[...] = acc

    return pl.pallas_call(
        body,
        grid=(_N // _BM,),
        in_specs=[
            pl.BlockSpec((_BM, _D), lambda i: (i, 0)),
            pl.BlockSpec((_NC, _BM, _D), lambda i: (0, i, 0)),
            pl.BlockSpec((_NC, _BM, 1), lambda i: (0, i, 0)),
            pl.BlockSpec((_D, _D), lambda i: (0, 0)),
            pl.BlockSpec((1, _D), lambda i: (0, 0)),
            pl.BlockSpec((_D, _D), lambda i: (0, 0)),
        ],
        out_specs=pl.BlockSpec((_BM, _D), lambda i: (i, 0)),
        out_shape=jax.ShapeDtypeStruct((_N, _D), _F32),
    )(xin, p, cnt, Wl, bl, Wr)


def kernel(x, edge_index, Wl0, bl0, Wr0, Wl1, bl1, Wr1):
    src = edge_index[0].astype(jnp.int32)
    dst = edge_index[1].astype(jnp.int32)
    bl0r = bl0.reshape(1, _D)
    bl1r = bl1.reshape(1, _D)

    agg1, cnt_raw = _agg_with_count(x, src, dst)
    cnt = cnt_raw[:, :_N].reshape(_NC, _N, 1)
    h = _sage_tc(x, agg1[:, :_N], cnt, Wl0, bl0r, Wr0, relu=True)
    (agg2,) = _agg_no_count(h, src, dst)
    out = _sage_tc(h, agg2[:, :_N], cnt, Wl1, bl1r, Wr1, relu=False)
    return out
